# baseline (speedup 1.0000x reference)
import jax
import jax.numpy as jnp
from jax import lax
from jax.experimental import pallas as pl
from jax.experimental.pallas import tpu as pltpu
from jax.experimental.pallas import tpu_sc as plsc

N = 10000
E = 320000
F = 128
A = 16
EA = 16
H = 128

NC = 2
NS = 16
NW = NC * NS
C = 128
K = 4
EC = E // K
TT = EC // C
N_PAD = 10240
ROWS_PER_TILE = N_PAD // NS

R_NODE = 1000
R_EDGE = 1000


def _relu(x):
    return jnp.maximum(x, 0.0)


def _node_kernel(nf_ref, na_ref,
                 wd1, bd1, wd2, bd2, wd3, bd3, wd4, bd4,
                 wf1, bf1, wf2, bf2, wf3, bf3, wf4, bf4,
                 w1f, w1a, b1,
                 fcwa, fcwf, fcb,
                 h_ref, part_ref):
    nf = nf_ref[...]
    na = na_ref[...]

    a = _relu(jnp.dot(na, wd1[...], preferred_element_type=jnp.float32) + bd1[...])
    a = _relu(jnp.dot(a, wd2[...], preferred_element_type=jnp.float32) + bd2[...])
    a = _relu(jnp.dot(a, wd3[...], preferred_element_type=jnp.float32) + bd3[...])
    dst_attr = jnp.dot(a, wd4[...], preferred_element_type=jnp.float32) + bd4[...]

    f = _relu(jnp.dot(nf, wf1[...], preferred_element_type=jnp.float32) + bf1[...])
    f = _relu(jnp.dot(f, wf2[...], preferred_element_type=jnp.float32) + bf2[...])
    f = _relu(jnp.dot(f, wf3[...], preferred_element_type=jnp.float32) + bf3[...])
    dst_feat = jnp.dot(f, wf4[...], preferred_element_type=jnp.float32) + bf4[...]

    h_ref[...] = (jnp.dot(nf, w1f[...], preferred_element_type=jnp.float32)
                  + jnp.dot(na, w1a[...], preferred_element_type=jnp.float32)
                  + b1[...])
    part_ref[...] = (jnp.dot(dst_attr, fcwa[...], preferred_element_type=jnp.float32)
                     + jnp.dot(dst_feat, fcwf[...], preferred_element_type=jnp.float32)
                     + fcb[...])


def _full(shape):
    return pl.BlockSpec(shape, lambda i: tuple(0 for _ in shape))


def _gather_body(table, idx, out, idx_v, rows_v, sem):
    c = lax.axis_index("c")
    s = lax.axis_index("s")
    wid = s * NC + c
    n = TT // NW + jnp.where(wid < TT % NW, 1, 0)

    def body(j, carry):
        t = (j * NW + wid) * C
        pltpu.sync_copy(idx.at[pl.ds(t, C)], idx_v)
        pltpu.async_copy(table.at[idx_v], rows_v, sem).wait()
        pltpu.sync_copy(rows_v, out.at[pl.ds(t, C)])
        return carry

    lax.fori_loop(0, n, body, 0)


def _scatter_body(feat, dsti, zeros, out, idx_v, feat_v, sem, acc_sh):
    c = lax.axis_index("c")
    s = lax.axis_index("s")
    wid = s * NC + c
    del sem
    n = TT // NW + jnp.where(wid < TT % NW, 1, 0)

    row0 = s * ROWS_PER_TILE
    pltpu.sync_copy(zeros.at[pl.ds(row0, ROWS_PER_TILE)],
                    acc_sh.at[pl.ds(row0, ROWS_PER_TILE)])
    plsc.subcore_barrier()

    def body(j, carry):
        t = (j * NW + wid) * C
        pltpu.sync_copy(dsti.at[pl.ds(t, C)], idx_v)
        pltpu.sync_copy(feat.at[pl.ds(t, C)], feat_v)
        pltpu.sync_copy(feat_v, acc_sh.at[idx_v], add=True)
        return carry

    lax.fori_loop(0, n, body, 0)
    plsc.subcore_barrier()
    pltpu.sync_copy(acc_sh.at[pl.ds(row0, ROWS_PER_TILE)],
                    out.at[c, pl.ds(row0, ROWS_PER_TILE)])


def _edge_kernel(h_ref, ea_ref, w1e, w2, b2, w3, b3, w4, b4, o_ref):
    eproj = jnp.dot(ea_ref[...].astype(jnp.bfloat16), w1e[...],
                    preferred_element_type=jnp.float32)
    x = _relu(h_ref[...] + eproj).astype(jnp.bfloat16)
    x = _relu(jnp.dot(x, w2[...], preferred_element_type=jnp.float32)
              + b2[...]).astype(jnp.bfloat16)
    x = _relu(jnp.dot(x, w3[...], preferred_element_type=jnp.float32)
              + b3[...]).astype(jnp.bfloat16)
    o_ref[...] = jnp.dot(x, w4[...], preferred_element_type=jnp.float32) + b4[...]


def _final_kernel(part_ref, *refs):
    agg_refs = refs[:-2]
    fcwg = refs[-2]
    o_ref = refs[-1]
    agg = agg_refs[0][...]
    for r in agg_refs[1:]:
        agg = agg + r[...]
    o_ref[...] = part_ref[...] + jnp.dot(agg, fcwg[...],
                                         preferred_element_type=jnp.float32)


def kernel(edge_index, node_feat, node_attr, edge_attr, src_params, dst_params,
           feat_params, fc_W, fc_b):
    src = edge_index[0]
    dst = edge_index[1]

    (ws1, bs1), (ws2, bs2), (ws3, bs3), (ws4, bs4) = src_params
    w1f = ws1[:F]
    w1a = ws1[F:F + A]
    w1e = ws1[F + A:]

    def row(b):
        return b.reshape(1, -1)

    grid_n = N // R_NODE
    wd, bd = zip(*dst_params)
    wf, bf = zip(*feat_params)
    node_in = [node_feat, node_attr,
               wd[0], row(bd[0]), wd[1], row(bd[1]), wd[2], row(bd[2]), wd[3], row(bd[3]),
               wf[0], row(bf[0]), wf[1], row(bf[1]), wf[2], row(bf[2]), wf[3], row(bf[3]),
               w1f, w1a, row(bs1),
               fc_W[:F], fc_W[F:2 * F], row(fc_b)]
    node_specs = [pl.BlockSpec((R_NODE, F), lambda i: (i, 0)),
                  pl.BlockSpec((R_NODE, A), lambda i: (i, 0))]
    node_specs += [_full(x.shape) for x in node_in[2:]]
    h_node, partial = pl.pallas_call(
        _node_kernel,
        grid=(grid_n,),
        in_specs=node_specs,
        out_specs=[pl.BlockSpec((R_NODE, F), lambda i: (i, 0)),
                   pl.BlockSpec((R_NODE, F), lambda i: (i, 0))],
        out_shape=[jax.ShapeDtypeStruct((N, F), jnp.float32),
                   jax.ShapeDtypeStruct((N, F), jnp.float32)],
    )(*node_in)

    mesh = plsc.VectorSubcoreMesh(core_axis_name="c", subcore_axis_name="s")
    gather_fn = pl.kernel(
        _gather_body,
        out_type=jax.ShapeDtypeStruct((EC, F), jnp.float32),
        mesh=mesh,
        scratch_types=[
            pltpu.VMEM((C,), jnp.int32),
            pltpu.VMEM((C, F), jnp.float32),
            pltpu.SemaphoreType.DMA,
        ],
    )
    scatter_fn = pl.kernel(
        _scatter_body,
        out_type=jax.ShapeDtypeStruct((NC, N_PAD, F), jnp.float32),
        mesh=mesh,
        scratch_types=[
            pltpu.VMEM((C,), jnp.int32),
            pltpu.VMEM((C, F), jnp.float32),
            pltpu.SemaphoreType.DMA,
            pltpu.VMEM_SHARED((N_PAD, F), jnp.float32),
        ],
    )

    bf16 = jnp.bfloat16
    edge_weights = [w1e.astype(bf16), ws2.astype(bf16), row(bs2),
                    ws3.astype(bf16), row(bs3), ws4.astype(bf16), row(bs4)]
    grid_e = EC // R_EDGE
    edge_specs = [pl.BlockSpec((R_EDGE, F), lambda i: (i, 0)),
                  pl.BlockSpec((R_EDGE, EA), lambda i: (i, 0))]
    edge_specs += [_full(x.shape) for x in edge_weights]
    edge_mlp = pl.pallas_call(
        _edge_kernel,
        grid=(grid_e,),
        in_specs=edge_specs,
        out_specs=pl.BlockSpec((R_EDGE, F), lambda i: (i, 0)),
        out_shape=jax.ShapeDtypeStruct((EC, F), jnp.float32),
    )

    zeros = jnp.zeros((N_PAD, F), jnp.float32)
    aggs = []
    for k in range(K):
        src_k = lax.dynamic_slice(src, (k * EC,), (EC,))
        dst_k = lax.dynamic_slice(dst, (k * EC,), (EC,))
        ea_k = lax.dynamic_slice(edge_attr, (k * EC, 0), (EC, EA))
        h_src_k = gather_fn(h_node, src_k)
        feat_k = edge_mlp(h_src_k, ea_k, *edge_weights)
        agg_k = scatter_fn(feat_k, dst_k, zeros)
        aggs.append(agg_k[0])
        aggs.append(agg_k[1])

    out = pl.pallas_call(
        _final_kernel,
        grid=(grid_n,),
        in_specs=([pl.BlockSpec((R_NODE, F), lambda i: (i, 0))]
                  * (1 + len(aggs)) + [_full((F, F))]),
        out_specs=pl.BlockSpec((R_NODE, F), lambda i: (i, 0)),
        out_shape=jax.ShapeDtypeStruct((N, F), jnp.float32),
    )(partial, *aggs, fc_W[2 * F:])
    return out


# speedup vs baseline: 1.0151x; 1.0151x over previous
import jax
import jax.numpy as jnp
from jax import lax
from jax.experimental import pallas as pl
from jax.experimental.pallas import tpu as pltpu
from jax.experimental.pallas import tpu_sc as plsc

N = 10000
E = 320000
F = 128
A = 16
EA = 16
H = 128

NC = 2
NS = 16
NW = NC * NS
C = 128
K = 5
EC = E // K
NCH_MAX = -(-(EC // C) // NW)
TT = EC // C
N_PAD = 10240
ROWS_PER_TILE = N_PAD // NS

R_NODE = 1000
R_EDGE = 1000


def _relu(x):
    return jnp.maximum(x, 0.0)


def _node_kernel(nf_ref, na_ref,
                 wd1, bd1, wd2, bd2, wd3, bd3, wd4, bd4,
                 wf1, bf1, wf2, bf2, wf3, bf3, wf4, bf4,
                 w1f, w1a, b1,
                 fcwa, fcwf, fcb,
                 h_ref, part_ref):
    nf = nf_ref[...]
    na = na_ref[...]

    a = _relu(jnp.dot(na, wd1[...], preferred_element_type=jnp.float32) + bd1[...])
    a = _relu(jnp.dot(a, wd2[...], preferred_element_type=jnp.float32) + bd2[...])
    a = _relu(jnp.dot(a, wd3[...], preferred_element_type=jnp.float32) + bd3[...])
    dst_attr = jnp.dot(a, wd4[...], preferred_element_type=jnp.float32) + bd4[...]

    f = _relu(jnp.dot(nf, wf1[...], preferred_element_type=jnp.float32) + bf1[...])
    f = _relu(jnp.dot(f, wf2[...], preferred_element_type=jnp.float32) + bf2[...])
    f = _relu(jnp.dot(f, wf3[...], preferred_element_type=jnp.float32) + bf3[...])
    dst_feat = jnp.dot(f, wf4[...], preferred_element_type=jnp.float32) + bf4[...]

    h_ref[...] = (jnp.dot(nf, w1f[...], preferred_element_type=jnp.float32)
                  + jnp.dot(na, w1a[...], preferred_element_type=jnp.float32)
                  + b1[...])
    part_ref[...] = (jnp.dot(dst_attr, fcwa[...], preferred_element_type=jnp.float32)
                     + jnp.dot(dst_feat, fcwf[...], preferred_element_type=jnp.float32)
                     + fcb[...])


def _full(shape):
    return pl.BlockSpec(shape, lambda i: tuple(0 for _ in shape))


def _prefetch_idx(idx_hbm, idx2d, wid, sem):
    handles = []
    for j in range(NCH_MAX):
        t = jnp.minimum(j * NW + wid, TT - 1) * C
        handles.append(pltpu.async_copy(idx_hbm.at[pl.ds(t, C)],
                                        idx2d.at[j], sem))
    for h in handles:
        h.wait()


def _gather_body(table, idx, out, idx2d, rows_v, isem, gsem):
    c = lax.axis_index("c")
    s = lax.axis_index("s")
    wid = s * NC + c
    n = TT // NW + jnp.where(wid < TT % NW, 1, 0)
    _prefetch_idx(idx, idx2d, wid, isem)

    def body(j, carry):
        t = (j * NW + wid) * C
        pltpu.async_copy(table.at[idx2d.at[j]], rows_v, gsem).wait()
        pltpu.sync_copy(rows_v, out.at[pl.ds(t, C)])
        return carry

    lax.fori_loop(0, n, body, 0)


def _scatter_body(feat, dsti, zeros, out, idx2d, feat_v, isem, acc_sh):
    c = lax.axis_index("c")
    s = lax.axis_index("s")
    wid = s * NC + c
    n = TT // NW + jnp.where(wid < TT % NW, 1, 0)

    row0 = s * ROWS_PER_TILE
    pltpu.sync_copy(zeros.at[pl.ds(row0, ROWS_PER_TILE)],
                    acc_sh.at[pl.ds(row0, ROWS_PER_TILE)])
    _prefetch_idx(dsti, idx2d, wid, isem)
    plsc.subcore_barrier()

    def body(j, carry):
        t = (j * NW + wid) * C
        pltpu.sync_copy(feat.at[pl.ds(t, C)], feat_v)
        pltpu.sync_copy(feat_v, acc_sh.at[idx2d.at[j]], add=True)
        return carry

    lax.fori_loop(0, n, body, 0)
    plsc.subcore_barrier()
    pltpu.sync_copy(acc_sh.at[pl.ds(row0, ROWS_PER_TILE)],
                    out.at[c, pl.ds(row0, ROWS_PER_TILE)])


def _edge_kernel(h_ref, ea_ref, w1e, w2, b2, w3, b3, w4, b4, o_ref):
    eproj = jnp.dot(ea_ref[...].astype(jnp.bfloat16), w1e[...],
                    preferred_element_type=jnp.float32)
    x = _relu(h_ref[...] + eproj).astype(jnp.bfloat16)
    x = _relu(jnp.dot(x, w2[...], preferred_element_type=jnp.float32)
              + b2[...]).astype(jnp.bfloat16)
    x = _relu(jnp.dot(x, w3[...], preferred_element_type=jnp.float32)
              + b3[...]).astype(jnp.bfloat16)
    o_ref[...] = jnp.dot(x, w4[...], preferred_element_type=jnp.float32) + b4[...]


def _final_kernel(part_ref, *refs):
    agg_refs = refs[:-2]
    fcwg = refs[-2]
    o_ref = refs[-1]
    agg = agg_refs[0][...]
    for r in agg_refs[1:]:
        agg = agg + r[...]
    o_ref[...] = part_ref[...] + jnp.dot(agg, fcwg[...],
                                         preferred_element_type=jnp.float32)


def kernel(edge_index, node_feat, node_attr, edge_attr, src_params, dst_params,
           feat_params, fc_W, fc_b):
    src = edge_index[0]
    dst = edge_index[1]

    (ws1, bs1), (ws2, bs2), (ws3, bs3), (ws4, bs4) = src_params
    w1f = ws1[:F]
    w1a = ws1[F:F + A]
    w1e = ws1[F + A:]

    def row(b):
        return b.reshape(1, -1)

    grid_n = N // R_NODE
    wd, bd = zip(*dst_params)
    wf, bf = zip(*feat_params)
    node_in = [node_feat, node_attr,
               wd[0], row(bd[0]), wd[1], row(bd[1]), wd[2], row(bd[2]), wd[3], row(bd[3]),
               wf[0], row(bf[0]), wf[1], row(bf[1]), wf[2], row(bf[2]), wf[3], row(bf[3]),
               w1f, w1a, row(bs1),
               fc_W[:F], fc_W[F:2 * F], row(fc_b)]
    node_specs = [pl.BlockSpec((R_NODE, F), lambda i: (i, 0)),
                  pl.BlockSpec((R_NODE, A), lambda i: (i, 0))]
    node_specs += [_full(x.shape) for x in node_in[2:]]
    h_node, partial = pl.pallas_call(
        _node_kernel,
        grid=(grid_n,),
        in_specs=node_specs,
        out_specs=[pl.BlockSpec((R_NODE, F), lambda i: (i, 0)),
                   pl.BlockSpec((R_NODE, F), lambda i: (i, 0))],
        out_shape=[jax.ShapeDtypeStruct((N, F), jnp.float32),
                   jax.ShapeDtypeStruct((N, F), jnp.float32)],
    )(*node_in)

    mesh = plsc.VectorSubcoreMesh(core_axis_name="c", subcore_axis_name="s")
    gather_fn = pl.kernel(
        _gather_body,
        out_type=jax.ShapeDtypeStruct((EC, F), jnp.float32),
        mesh=mesh,
        scratch_types=[
            pltpu.VMEM((NCH_MAX, C), jnp.int32),
            pltpu.VMEM((C, F), jnp.float32),
            pltpu.SemaphoreType.DMA,
            pltpu.SemaphoreType.DMA,
        ],
    )
    scatter_fn = pl.kernel(
        _scatter_body,
        out_type=jax.ShapeDtypeStruct((NC, N_PAD, F), jnp.float32),
        mesh=mesh,
        scratch_types=[
            pltpu.VMEM((NCH_MAX, C), jnp.int32),
            pltpu.VMEM((C, F), jnp.float32),
            pltpu.SemaphoreType.DMA,
            pltpu.VMEM_SHARED((N_PAD, F), jnp.float32),
        ],
    )

    bf16 = jnp.bfloat16
    edge_weights = [w1e.astype(bf16), ws2.astype(bf16), row(bs2),
                    ws3.astype(bf16), row(bs3), ws4.astype(bf16), row(bs4)]
    grid_e = EC // R_EDGE
    edge_specs = [pl.BlockSpec((R_EDGE, F), lambda i: (i, 0)),
                  pl.BlockSpec((R_EDGE, EA), lambda i: (i, 0))]
    edge_specs += [_full(x.shape) for x in edge_weights]
    edge_mlp = pl.pallas_call(
        _edge_kernel,
        grid=(grid_e,),
        in_specs=edge_specs,
        out_specs=pl.BlockSpec((R_EDGE, F), lambda i: (i, 0)),
        out_shape=jax.ShapeDtypeStruct((EC, F), jnp.float32),
    )

    zeros = jnp.zeros((N_PAD, F), jnp.float32)
    aggs = []
    for k in range(K):
        src_k = lax.dynamic_slice(src, (k * EC,), (EC,))
        dst_k = lax.dynamic_slice(dst, (k * EC,), (EC,))
        ea_k = lax.dynamic_slice(edge_attr, (k * EC, 0), (EC, EA))
        h_src_k = gather_fn(h_node, src_k)
        feat_k = edge_mlp(h_src_k, ea_k, *edge_weights)
        agg_k = scatter_fn(feat_k, dst_k, zeros)
        aggs.append(agg_k[0])
        aggs.append(agg_k[1])

    out = pl.pallas_call(
        _final_kernel,
        grid=(grid_n,),
        in_specs=([pl.BlockSpec((R_NODE, F), lambda i: (i, 0))]
                  * (1 + len(aggs)) + [_full((F, F))]),
        out_specs=pl.BlockSpec((R_NODE, F), lambda i: (i, 0)),
        out_shape=jax.ShapeDtypeStruct((N, F), jnp.float32),
    )(partial, *aggs, fc_W[2 * F:])
    return out


# speedup vs baseline: 1.0521x; 1.0365x over previous
import jax
import jax.numpy as jnp
from jax import lax
from jax.experimental import pallas as pl
from jax.experimental.pallas import tpu as pltpu
from jax.experimental.pallas import tpu_sc as plsc

N = 10000
E = 320000
F = 128
A = 16
EA = 16
H = 128

NC = 2
NS = 16
NW = NC * NS
C = 128
K = 5
EC = E // K
NCH_MAX = -(-(EC // C) // NW)
TT = EC // C
N_PAD = 10240
ROWS_PER_TILE = N_PAD // NS

R_NODE = 1024
R_FIN = 1000
R_EDGE = 1000


def _relu(x):
    return jnp.maximum(x, 0.0)


def _node_kernel(nf_ref, na_ref,
                 wd1, bd1, wd2, bd2, wd3, bd3, wd4, bd4,
                 wf1, bf1, wf2, bf2, wf3, bf3, wf4, bf4,
                 w1f, w1a, b1,
                 fcwa, fcwf, fcb,
                 h_ref, part_ref):
    nf = nf_ref[...]
    na = na_ref[...]

    a = _relu(jnp.dot(na, wd1[...], preferred_element_type=jnp.float32) + bd1[...])
    a = _relu(jnp.dot(a, wd2[...], preferred_element_type=jnp.float32) + bd2[...])
    a = _relu(jnp.dot(a, wd3[...], preferred_element_type=jnp.float32) + bd3[...])
    dst_attr = jnp.dot(a, wd4[...], preferred_element_type=jnp.float32) + bd4[...]

    f = _relu(jnp.dot(nf, wf1[...], preferred_element_type=jnp.float32) + bf1[...])
    f = _relu(jnp.dot(f, wf2[...], preferred_element_type=jnp.float32) + bf2[...])
    f = _relu(jnp.dot(f, wf3[...], preferred_element_type=jnp.float32) + bf3[...])
    dst_feat = jnp.dot(f, wf4[...], preferred_element_type=jnp.float32) + bf4[...]

    h_ref[...] = (jnp.dot(nf, w1f[...], preferred_element_type=jnp.float32)
                  + jnp.dot(na, w1a[...], preferred_element_type=jnp.float32)
                  + b1[...])
    part_ref[...] = (jnp.dot(dst_attr, fcwa[...], preferred_element_type=jnp.float32)
                     + jnp.dot(dst_feat, fcwf[...], preferred_element_type=jnp.float32)
                     + fcb[...])


def _full(shape):
    return pl.BlockSpec(shape, lambda i: tuple(0 for _ in shape))


def _prefetch_idx(idx_hbm, idx2d, wid, sem):
    handles = []
    for j in range(NCH_MAX):
        t = jnp.minimum(j * NW + wid, TT - 1) * C
        handles.append(pltpu.async_copy(idx_hbm.at[pl.ds(t, C)],
                                        idx2d.at[j], sem))
    for h in handles:
        h.wait()


def _gather_body(table, idx, out, idx2d, rows_v, isem, gsem, tbl_sh):
    c = lax.axis_index("c")
    s = lax.axis_index("s")
    wid = s * NC + c
    n = TT // NW + jnp.where(wid < TT % NW, 1, 0)
    row0 = s * ROWS_PER_TILE
    pltpu.sync_copy(table.at[pl.ds(row0, ROWS_PER_TILE)],
                    tbl_sh.at[pl.ds(row0, ROWS_PER_TILE)])
    _prefetch_idx(idx, idx2d, wid, isem)
    plsc.subcore_barrier()

    def body(j, carry):
        t = (j * NW + wid) * C
        pltpu.async_copy(tbl_sh.at[idx2d.at[j]], rows_v, gsem).wait()
        pltpu.sync_copy(rows_v, out.at[pl.ds(t, C)])
        return carry

    lax.fori_loop(0, n, body, 0)


def _scatter_body(feat, dsti, zeros, out, idx2d, feat_v, isem, acc_sh):
    c = lax.axis_index("c")
    s = lax.axis_index("s")
    wid = s * NC + c
    n = TT // NW + jnp.where(wid < TT % NW, 1, 0)

    row0 = s * ROWS_PER_TILE
    pltpu.sync_copy(zeros.at[pl.ds(row0, ROWS_PER_TILE)],
                    acc_sh.at[pl.ds(row0, ROWS_PER_TILE)])
    _prefetch_idx(dsti, idx2d, wid, isem)
    plsc.subcore_barrier()

    def body(j, carry):
        t = (j * NW + wid) * C
        pltpu.sync_copy(feat.at[pl.ds(t, C)], feat_v)
        pltpu.sync_copy(feat_v, acc_sh.at[idx2d.at[j]], add=True)
        return carry

    lax.fori_loop(0, n, body, 0)
    plsc.subcore_barrier()
    pltpu.sync_copy(acc_sh.at[pl.ds(row0, ROWS_PER_TILE)],
                    out.at[c, pl.ds(row0, ROWS_PER_TILE)])


def _edge_kernel(h_ref, ea_ref, w1e, w2, b2, w3, b3, w4, b4, o_ref):
    eproj = jnp.dot(ea_ref[...].astype(jnp.bfloat16), w1e[...],
                    preferred_element_type=jnp.float32)
    x = _relu(h_ref[...] + eproj).astype(jnp.bfloat16)
    x = _relu(jnp.dot(x, w2[...], preferred_element_type=jnp.float32)
              + b2[...]).astype(jnp.bfloat16)
    x = _relu(jnp.dot(x, w3[...], preferred_element_type=jnp.float32)
              + b3[...]).astype(jnp.bfloat16)
    o_ref[...] = jnp.dot(x, w4[...], preferred_element_type=jnp.float32) + b4[...]


def _final_kernel(part_ref, *refs):
    agg_refs = refs[:-2]
    fcwg = refs[-2]
    o_ref = refs[-1]
    agg = agg_refs[0][...]
    for r in agg_refs[1:]:
        agg = agg + r[...]
    o_ref[...] = part_ref[...] + jnp.dot(agg, fcwg[...],
                                         preferred_element_type=jnp.float32)


def kernel(edge_index, node_feat, node_attr, edge_attr, src_params, dst_params,
           feat_params, fc_W, fc_b):
    src = edge_index[0]
    dst = edge_index[1]

    (ws1, bs1), (ws2, bs2), (ws3, bs3), (ws4, bs4) = src_params
    w1f = ws1[:F]
    w1a = ws1[F:F + A]
    w1e = ws1[F + A:]

    def row(b):
        return b.reshape(1, -1)

    grid_n = N_PAD // R_NODE
    wd, bd = zip(*dst_params)
    wf, bf = zip(*feat_params)
    node_in = [node_feat, node_attr,
               wd[0], row(bd[0]), wd[1], row(bd[1]), wd[2], row(bd[2]), wd[3], row(bd[3]),
               wf[0], row(bf[0]), wf[1], row(bf[1]), wf[2], row(bf[2]), wf[3], row(bf[3]),
               w1f, w1a, row(bs1),
               fc_W[:F], fc_W[F:2 * F], row(fc_b)]
    node_specs = [pl.BlockSpec((R_NODE, F), lambda i: (i, 0)),
                  pl.BlockSpec((R_NODE, A), lambda i: (i, 0))]
    node_specs += [_full(x.shape) for x in node_in[2:]]
    h_node, partial = pl.pallas_call(
        _node_kernel,
        grid=(grid_n,),
        in_specs=node_specs,
        out_specs=[pl.BlockSpec((R_NODE, F), lambda i: (i, 0)),
                   pl.BlockSpec((R_NODE, F), lambda i: (i, 0))],
        out_shape=[jax.ShapeDtypeStruct((N_PAD, F), jnp.float32),
                   jax.ShapeDtypeStruct((N_PAD, F), jnp.float32)],
    )(*node_in)

    mesh = plsc.VectorSubcoreMesh(core_axis_name="c", subcore_axis_name="s")
    gather_fn = pl.kernel(
        _gather_body,
        out_type=jax.ShapeDtypeStruct((EC, F), jnp.float32),
        mesh=mesh,
        scratch_types=[
            pltpu.VMEM((NCH_MAX, C), jnp.int32),
            pltpu.VMEM((C, F), jnp.float32),
            pltpu.SemaphoreType.DMA,
            pltpu.SemaphoreType.DMA,
            pltpu.VMEM_SHARED((N_PAD, F), jnp.float32),
        ],
    )
    scatter_fn = pl.kernel(
        _scatter_body,
        out_type=jax.ShapeDtypeStruct((NC, N_PAD, F), jnp.float32),
        mesh=mesh,
        scratch_types=[
            pltpu.VMEM((NCH_MAX, C), jnp.int32),
            pltpu.VMEM((C, F), jnp.float32),
            pltpu.SemaphoreType.DMA,
            pltpu.VMEM_SHARED((N_PAD, F), jnp.float32),
        ],
    )

    bf16 = jnp.bfloat16
    edge_weights = [w1e.astype(bf16), ws2.astype(bf16), row(bs2),
                    ws3.astype(bf16), row(bs3), ws4.astype(bf16), row(bs4)]
    grid_e = EC // R_EDGE
    edge_specs = [pl.BlockSpec((R_EDGE, F), lambda i: (i, 0)),
                  pl.BlockSpec((R_EDGE, EA), lambda i: (i, 0))]
    edge_specs += [_full(x.shape) for x in edge_weights]
    edge_mlp = pl.pallas_call(
        _edge_kernel,
        grid=(grid_e,),
        in_specs=edge_specs,
        out_specs=pl.BlockSpec((R_EDGE, F), lambda i: (i, 0)),
        out_shape=jax.ShapeDtypeStruct((EC, F), jnp.float32),
    )

    zeros = jnp.zeros((N_PAD, F), jnp.float32)
    aggs = []
    for k in range(K):
        src_k = lax.dynamic_slice(src, (k * EC,), (EC,))
        dst_k = lax.dynamic_slice(dst, (k * EC,), (EC,))
        ea_k = lax.dynamic_slice(edge_attr, (k * EC, 0), (EC, EA))
        h_src_k = gather_fn(h_node, src_k)
        feat_k = edge_mlp(h_src_k, ea_k, *edge_weights)
        agg_k = scatter_fn(feat_k, dst_k, zeros)
        aggs.append(agg_k[0])
        aggs.append(agg_k[1])

    out = pl.pallas_call(
        _final_kernel,
        grid=(N // R_FIN,),
        in_specs=([pl.BlockSpec((R_FIN, F), lambda i: (i, 0))]
                  * (1 + len(aggs)) + [_full((F, F))]),
        out_specs=pl.BlockSpec((R_FIN, F), lambda i: (i, 0)),
        out_shape=jax.ShapeDtypeStruct((N, F), jnp.float32),
    )(partial, *aggs, fc_W[2 * F:])
    return out


# speedup vs baseline: 1.0545x; 1.0022x over previous
import jax
import jax.numpy as jnp
from jax import lax
from jax.experimental import pallas as pl
from jax.experimental.pallas import tpu as pltpu
from jax.experimental.pallas import tpu_sc as plsc

N = 10000
E = 320000
F = 128
A = 16
EA = 16
H = 128

NC = 2
NS = 16
NW = NC * NS
C = 128
K = 5
EC = E // K
NCH_MAX = -(-(EC // C) // NW)
TT = EC // C
N_PAD = 10240
ROWS_PER_TILE = N_PAD // NS

R_NODE = 1024
R_FIN = 1000
R_EDGE = 1000


def _relu(x):
    return jnp.maximum(x, 0.0)


def _node_kernel(nf_ref, na_ref,
                 wd1, bd1, wd2, bd2, wd3, bd3, wd4, bd4,
                 wf1, bf1, wf2, bf2, wf3, bf3, wf4, bf4,
                 w1f, w1a, b1,
                 fcwa, fcwf, fcb,
                 h_ref, part_ref):
    nf = nf_ref[...]
    na = na_ref[...]

    a = _relu(jnp.dot(na, wd1[...], preferred_element_type=jnp.float32) + bd1[...])
    a = _relu(jnp.dot(a, wd2[...], preferred_element_type=jnp.float32) + bd2[...])
    a = _relu(jnp.dot(a, wd3[...], preferred_element_type=jnp.float32) + bd3[...])
    dst_attr = jnp.dot(a, wd4[...], preferred_element_type=jnp.float32) + bd4[...]

    f = _relu(jnp.dot(nf, wf1[...], preferred_element_type=jnp.float32) + bf1[...])
    f = _relu(jnp.dot(f, wf2[...], preferred_element_type=jnp.float32) + bf2[...])
    f = _relu(jnp.dot(f, wf3[...], preferred_element_type=jnp.float32) + bf3[...])
    dst_feat = jnp.dot(f, wf4[...], preferred_element_type=jnp.float32) + bf4[...]

    h_ref[...] = (jnp.dot(nf, w1f[...], preferred_element_type=jnp.float32)
                  + jnp.dot(na, w1a[...], preferred_element_type=jnp.float32)
                  + b1[...])
    part_ref[...] = (jnp.dot(dst_attr, fcwa[...], preferred_element_type=jnp.float32)
                     + jnp.dot(dst_feat, fcwf[...], preferred_element_type=jnp.float32)
                     + fcb[...])


def _full(shape):
    return pl.BlockSpec(shape, lambda i: tuple(0 for _ in shape))


def _prefetch_idx(idx_hbm, idx2d, wid, sem):
    handles = []
    for j in range(NCH_MAX):
        t = jnp.minimum(j * NW + wid, TT - 1) * C
        handles.append(pltpu.async_copy(idx_hbm.at[pl.ds(t, C)],
                                        idx2d.at[j], sem))
    for h in handles:
        h.wait()


def _gather_body(table, idx, out, idx2d, rows0, rows1, isem, g0, g1, s0, s1,
                 tbl_sh):
    c = lax.axis_index("c")
    s = lax.axis_index("s")
    wid = s * NC + c
    n = TT // NW + jnp.where(wid < TT % NW, 1, 0)
    n_static = TT // NW
    rows = (rows0, rows1)
    gsem = (g0, g1)
    ssem = (s0, s1)
    row0 = s * ROWS_PER_TILE
    pltpu.sync_copy(table.at[pl.ds(row0, ROWS_PER_TILE)],
                    tbl_sh.at[pl.ds(row0, ROWS_PER_TILE)])
    _prefetch_idx(idx, idx2d, wid, isem)
    plsc.subcore_barrier()

    def t_off(j):
        return (j * NW + wid) * C

    hg = [None] * n_static
    hs = [None] * n_static
    for j in range(n_static):
        b = j % 2
        if j >= 1:
            hg[j - 1].wait()
            hs[j - 1] = pltpu.async_copy(
                rows[1 - b], out.at[pl.ds(t_off(j - 1), C)], ssem[1 - b])
        if j >= 2:
            hs[j - 2].wait()
        hg[j] = pltpu.async_copy(tbl_sh.at[idx2d.at[j]], rows[b], gsem[b])
    last = n_static - 1
    hg[last].wait()
    hs[last] = pltpu.async_copy(
        rows[last % 2], out.at[pl.ds(t_off(last), C)], ssem[last % 2])
    hs[last - 1].wait()
    hs[last].wait()

    def body(j, carry):
        pltpu.async_copy(tbl_sh.at[idx2d.at[j]], rows0, gsem[0]).wait()
        pltpu.sync_copy(rows0, out.at[pl.ds(t_off(j), C)])
        return carry

    lax.fori_loop(n_static, n, body, 0)


def _scatter_body(feat, dsti, zeros, out, idx2d, feat_v, isem, acc_sh):
    c = lax.axis_index("c")
    s = lax.axis_index("s")
    wid = s * NC + c
    n = TT // NW + jnp.where(wid < TT % NW, 1, 0)

    row0 = s * ROWS_PER_TILE
    pltpu.sync_copy(zeros.at[pl.ds(row0, ROWS_PER_TILE)],
                    acc_sh.at[pl.ds(row0, ROWS_PER_TILE)])
    _prefetch_idx(dsti, idx2d, wid, isem)
    plsc.subcore_barrier()

    def body(j, carry):
        t = (j * NW + wid) * C
        pltpu.sync_copy(feat.at[pl.ds(t, C)], feat_v)
        pltpu.sync_copy(feat_v, acc_sh.at[idx2d.at[j]], add=True)
        return carry

    lax.fori_loop(0, n, body, 0)
    plsc.subcore_barrier()
    pltpu.sync_copy(acc_sh.at[pl.ds(row0, ROWS_PER_TILE)],
                    out.at[c, pl.ds(row0, ROWS_PER_TILE)])


def _edge_kernel(h_ref, ea_ref, w1e, w2, b2, w3, b3, w4, b4, o_ref):
    eproj = jnp.dot(ea_ref[...].astype(jnp.bfloat16), w1e[...],
                    preferred_element_type=jnp.float32)
    x = _relu(h_ref[...] + eproj).astype(jnp.bfloat16)
    x = _relu(jnp.dot(x, w2[...], preferred_element_type=jnp.float32)
              + b2[...]).astype(jnp.bfloat16)
    x = _relu(jnp.dot(x, w3[...], preferred_element_type=jnp.float32)
              + b3[...]).astype(jnp.bfloat16)
    o_ref[...] = jnp.dot(x, w4[...], preferred_element_type=jnp.float32) + b4[...]


def _final_kernel(part_ref, *refs):
    agg_refs = refs[:-2]
    fcwg = refs[-2]
    o_ref = refs[-1]
    agg = agg_refs[0][...]
    for r in agg_refs[1:]:
        agg = agg + r[...]
    o_ref[...] = part_ref[...] + jnp.dot(agg, fcwg[...],
                                         preferred_element_type=jnp.float32)


def kernel(edge_index, node_feat, node_attr, edge_attr, src_params, dst_params,
           feat_params, fc_W, fc_b):
    src = edge_index[0]
    dst = edge_index[1]

    (ws1, bs1), (ws2, bs2), (ws3, bs3), (ws4, bs4) = src_params
    w1f = ws1[:F]
    w1a = ws1[F:F + A]
    w1e = ws1[F + A:]

    def row(b):
        return b.reshape(1, -1)

    grid_n = N_PAD // R_NODE
    wd, bd = zip(*dst_params)
    wf, bf = zip(*feat_params)
    node_in = [node_feat, node_attr,
               wd[0], row(bd[0]), wd[1], row(bd[1]), wd[2], row(bd[2]), wd[3], row(bd[3]),
               wf[0], row(bf[0]), wf[1], row(bf[1]), wf[2], row(bf[2]), wf[3], row(bf[3]),
               w1f, w1a, row(bs1),
               fc_W[:F], fc_W[F:2 * F], row(fc_b)]
    node_specs = [pl.BlockSpec((R_NODE, F), lambda i: (i, 0)),
                  pl.BlockSpec((R_NODE, A), lambda i: (i, 0))]
    node_specs += [_full(x.shape) for x in node_in[2:]]
    h_node, partial = pl.pallas_call(
        _node_kernel,
        grid=(grid_n,),
        in_specs=node_specs,
        out_specs=[pl.BlockSpec((R_NODE, F), lambda i: (i, 0)),
                   pl.BlockSpec((R_NODE, F), lambda i: (i, 0))],
        out_shape=[jax.ShapeDtypeStruct((N_PAD, F), jnp.float32),
                   jax.ShapeDtypeStruct((N_PAD, F), jnp.float32)],
    )(*node_in)

    mesh = plsc.VectorSubcoreMesh(core_axis_name="c", subcore_axis_name="s")
    gather_fn = pl.kernel(
        _gather_body,
        out_type=jax.ShapeDtypeStruct((EC, F), jnp.float32),
        mesh=mesh,
        scratch_types=[
            pltpu.VMEM((NCH_MAX, C), jnp.int32),
            pltpu.VMEM((C, F), jnp.float32),
            pltpu.VMEM((C, F), jnp.float32),
            pltpu.SemaphoreType.DMA,
            pltpu.SemaphoreType.DMA,
            pltpu.SemaphoreType.DMA,
            pltpu.SemaphoreType.DMA,
            pltpu.SemaphoreType.DMA,
            pltpu.VMEM_SHARED((N_PAD, F), jnp.float32),
        ],
    )
    scatter_fn = pl.kernel(
        _scatter_body,
        out_type=jax.ShapeDtypeStruct((NC, N_PAD, F), jnp.float32),
        mesh=mesh,
        scratch_types=[
            pltpu.VMEM((NCH_MAX, C), jnp.int32),
            pltpu.VMEM((C, F), jnp.float32),
            pltpu.SemaphoreType.DMA,
            pltpu.VMEM_SHARED((N_PAD, F), jnp.float32),
        ],
    )

    bf16 = jnp.bfloat16
    edge_weights = [w1e.astype(bf16), ws2.astype(bf16), row(bs2),
                    ws3.astype(bf16), row(bs3), ws4.astype(bf16), row(bs4)]
    grid_e = EC // R_EDGE
    edge_specs = [pl.BlockSpec((R_EDGE, F), lambda i: (i, 0)),
                  pl.BlockSpec((R_EDGE, EA), lambda i: (i, 0))]
    edge_specs += [_full(x.shape) for x in edge_weights]
    edge_mlp = pl.pallas_call(
        _edge_kernel,
        grid=(grid_e,),
        in_specs=edge_specs,
        out_specs=pl.BlockSpec((R_EDGE, F), lambda i: (i, 0)),
        out_shape=jax.ShapeDtypeStruct((EC, F), jnp.float32),
    )

    zeros = jnp.zeros((N_PAD, F), jnp.float32)
    aggs = []
    for k in range(K):
        src_k = lax.dynamic_slice(src, (k * EC,), (EC,))
        dst_k = lax.dynamic_slice(dst, (k * EC,), (EC,))
        ea_k = lax.dynamic_slice(edge_attr, (k * EC, 0), (EC, EA))
        h_src_k = gather_fn(h_node, src_k)
        feat_k = edge_mlp(h_src_k, ea_k, *edge_weights)
        agg_k = scatter_fn(feat_k, dst_k, zeros)
        aggs.append(agg_k[0])
        aggs.append(agg_k[1])

    out = pl.pallas_call(
        _final_kernel,
        grid=(N // R_FIN,),
        in_specs=([pl.BlockSpec((R_FIN, F), lambda i: (i, 0))]
                  * (1 + len(aggs)) + [_full((F, F))]),
        out_specs=pl.BlockSpec((R_FIN, F), lambda i: (i, 0)),
        out_shape=jax.ShapeDtypeStruct((N, F), jnp.float32),
    )(partial, *aggs, fc_W[2 * F:])
    return out
